# Initial kernel scaffold; baseline (speedup 1.0000x reference)
import jax
import jax.numpy as jnp
from jax.experimental import pallas as pl


def kernel(x, edge_index, edge_type, Wk, bk, Wq, bq, Wv, bv, a_rel, m_rel, p_rel, Wo, bo, skip):
    raise NotImplementedError("write your pallas kernel here")


# speedup vs baseline: 5.7912x; 5.7912x over previous
import functools

import jax
import jax.numpy as jnp
import numpy as np
from jax import lax
from jax.experimental import pallas as pl
from jax.experimental.pallas import tpu as pltpu
from jax.experimental.pallas import tpu_sc as plsc

N = 10000
E = 320000
IN = 128
OUT = 128
H = 8
D = 16

CH = 64
NCH = E // CH
NW = 32
DROWS = N // 8
R = N + DROWS + 30
ZROWS = 40
NZB = R // ZROWS
EB = E // 10


def _front_body(x_ref, wq_ref, bq_ref, wk_ref, bk_ref, wv_ref, bv_ref,
                src_ref, et_ref, q_ref, k_ref, v_ref, kv_ref):
    xb = x_ref[...]
    dot = functools.partial(jnp.dot, precision=lax.Precision.HIGHEST,
                            preferred_element_type=jnp.float32)
    q_ref[...] = dot(xb, wq_ref[...]) + bq_ref[...]
    for r in range(2):
        k_ref[r] = dot(xb, wk_ref[r]) + bk_ref[r]
        v_ref[r] = dot(xb, wv_ref[r]) + bv_ref[r]
    s = src_ref[...]
    kv_ref[...] = jnp.where(et_ref[...] == 1, s, s + N)


def _front(x, Wq, bq, Wk2, bk2, Wv2, bv2, src2, et2):
    B = 1000
    nb = N // B
    return pl.pallas_call(
        _front_body,
        grid=(nb,),
        in_specs=[
            pl.BlockSpec((B, IN), lambda i: (i, 0)),
            pl.BlockSpec((IN, OUT), lambda i: (0, 0)),
            pl.BlockSpec((1, OUT), lambda i: (0, 0)),
            pl.BlockSpec((2, IN, OUT), lambda i: (0, 0, 0)),
            pl.BlockSpec((2, 1, OUT), lambda i: (0, 0, 0)),
            pl.BlockSpec((2, IN, OUT), lambda i: (0, 0, 0)),
            pl.BlockSpec((2, 1, OUT), lambda i: (0, 0, 0)),
            pl.BlockSpec((1, 1, EB), lambda i: (i, 0, 0)),
            pl.BlockSpec((1, 1, EB), lambda i: (i, 0, 0)),
        ],
        out_specs=[
            pl.BlockSpec((B, OUT), lambda i: (i, 0)),
            pl.BlockSpec((2, B, OUT), lambda i: (0, i, 0)),
            pl.BlockSpec((2, B, OUT), lambda i: (0, i, 0)),
            pl.BlockSpec((1, 1, EB), lambda i: (i, 0, 0)),
        ],
        out_shape=[
            jax.ShapeDtypeStruct((N, OUT), jnp.float32),
            jax.ShapeDtypeStruct((2, N, OUT), jnp.float32),
            jax.ShapeDtypeStruct((2, N, OUT), jnp.float32),
            jax.ShapeDtypeStruct((10, 1, EB), jnp.int32),
        ],
    )(x, Wq, bq, Wk2, bk2, Wv2, bv2, src2, et2)


def _sc_body(qt, kt, vt, dst, kvi, z64, oacc,
             qb, kb, vb, wb, dstb, kvb, dib, acc_sh, sem, sem2, sem3):
    cid = lax.axis_index("c")
    sid = lax.axis_index("s")
    wid = cid * 16 + sid
    i16 = lax.iota(jnp.int32, 16)
    zeros16 = jnp.zeros((16,), jnp.float32)

    def _zero_rows(i, _):
        for j in range(OUT // 16):
            vb[i, pl.ds(j * 16, 16)] = zeros16
        return 0
    lax.fori_loop(0, ZROWS, _zero_rows, 0)

    nz = jnp.where(sid < (NZB % 16), NZB // 16 + 1, NZB // 16)

    def _zstripe(i, _):
        r0 = (sid + i * 16) * ZROWS
        pltpu.sync_copy(vb.at[pl.ds(0, ZROWS)], acc_sh.at[pl.ds(r0, ZROWS)])
        return 0
    lax.fori_loop(0, nz, _zstripe, 0)
    plsc.subcore_barrier()

    nc = jnp.where(wid < (NCH % NW), NCH // NW + 1, NCH // NW)

    def _chunk(c, _):
        base = (wid + c * NW) * CH
        pltpu.sync_copy(dst.at[pl.ds(base, CH)], dstb)
        pltpu.sync_copy(kvi.at[pl.ds(base, CH)], kvb)

        cq = pltpu.async_copy(qt.at[dstb], qb, sem)
        ck = pltpu.async_copy(kt.at[kvb], kb, sem2)
        cv = pltpu.async_copy(vt.at[kvb], vb, sem3)
        cq.wait()
        ck.wait()

        def _alpha(gh, _):
            g = gh // H
            h = gh % H
            rows = g * 16 + i16
            acc = zeros16
            for d in range(D):
                col = jnp.broadcast_to(h * D + d, (16,)).astype(jnp.int32)
                qv = plsc.load_gather(qb, [rows, col])
                kv = plsc.load_gather(kb, [rows, col])
                acc = acc + qv * kv
            w16 = jnp.exp(acc)
            plsc.store_scatter(wb, [rows * 16 + h], w16)
            return 0
        lax.fori_loop(0, (CH // 16) * H, _alpha, 0)

        pltpu.sync_copy(z64, qb)

        def _den(g, _):
            rows = g * 16 + i16
            dst16 = dstb[pl.ds(g * 16, 16)]
            dib[pl.ds(g * 16, 16)] = N + (dst16 >> 3)
            cbase = (dst16 & 7) * 16
            for h in range(H):
                w16 = plsc.load_gather(wb, [rows * 16 + h])
                plsc.store_scatter(qb, [rows, cbase + h], w16)
            return 0
        lax.fori_loop(0, CH // 16, _den, 0)

        cv.wait()

        def _msg(e, _):
            for h in range(H):
                wspl = plsc.load_gather(wb, [jnp.broadcast_to(
                    e * 16 + h, (16,)).astype(jnp.int32)])
                v16 = vb[e, pl.ds(h * D, 16)]
                vb[e, pl.ds(h * D, 16)] = v16 * wspl
            return 0
        lax.fori_loop(0, CH, _msg, 0)

        pltpu.sync_copy(vb, acc_sh.at[dstb], add=True)
        pltpu.sync_copy(qb, acc_sh.at[dib], add=True)
        return 0

    lax.fori_loop(0, nc, _chunk, 0)
    plsc.subcore_barrier()

    def _fstripe(i, _):
        r0 = (sid + i * 16) * ZROWS
        pltpu.sync_copy(acc_sh.at[pl.ds(r0, ZROWS)],
                        oacc.at[cid, pl.ds(r0, ZROWS)])
        return 0
    lax.fori_loop(0, nz, _fstripe, 0)


@functools.partial(
    pl.kernel,
    out_type=jax.ShapeDtypeStruct((2, R, OUT), jnp.float32),
    mesh=plsc.VectorSubcoreMesh(core_axis_name="c", subcore_axis_name="s",
                                num_cores=2, num_subcores=16),
    compiler_params=pltpu.CompilerParams(needs_layout_passes=False),
    scratch_types=[
        pltpu.VMEM((CH, OUT), jnp.float32),
        pltpu.VMEM((CH, OUT), jnp.float32),
        pltpu.VMEM((CH, OUT), jnp.float32),
        pltpu.VMEM((CH * 16,), jnp.float32),
        pltpu.VMEM((CH,), jnp.int32),
        pltpu.VMEM((CH,), jnp.int32),
        pltpu.VMEM((CH,), jnp.int32),
        pltpu.VMEM_SHARED((R, OUT), jnp.float32),
        pltpu.SemaphoreType.DMA,
        pltpu.SemaphoreType.DMA,
        pltpu.SemaphoreType.DMA,
    ],
)
def _sc_edges(qt, kt, vt, dst, kvi, z64, oacc, *scratch):
    _sc_body(qt, kt, vt, dst, kvi, z64, oacc, *scratch)


def _back_body(n_ref, d_ref, x_ref, wo_ref, bo_ref, sk_ref, o_ref):
    num = n_ref[0] + n_ref[1]
    den = d_ref[0] + d_ref[1]
    rr = lax.broadcasted_iota(jnp.int32, (16, OUT), 0)
    cc = lax.broadcasted_iota(jnp.int32, (16, OUT), 1)
    expand = (cc // D == rr).astype(jnp.float32)
    den128 = jnp.dot(den, expand, precision=lax.Precision.HIGHEST,
                     preferred_element_type=jnp.float32)
    agg = num / (den128 + 1e-16)
    act = jax.nn.gelu(agg)
    out = jnp.dot(act, wo_ref[...], precision=lax.Precision.HIGHEST,
                  preferred_element_type=jnp.float32) + bo_ref[...]
    beta = jax.nn.sigmoid(sk_ref[0, 0])
    o_ref[...] = beta * out + (1.0 - beta) * x_ref[...]


def _back(onum, oden, x, Wo, bo, skip2):
    B = 1000
    nb = N // B
    return pl.pallas_call(
        _back_body,
        grid=(nb,),
        in_specs=[
            pl.BlockSpec((2, B, OUT), lambda i: (0, i, 0)),
            pl.BlockSpec((2, B, 16), lambda i: (0, i, 0)),
            pl.BlockSpec((B, IN), lambda i: (i, 0)),
            pl.BlockSpec((OUT, OUT), lambda i: (0, 0)),
            pl.BlockSpec((1, OUT), lambda i: (0, 0)),
            pl.BlockSpec((1, 1), lambda i: (0, 0)),
        ],
        out_specs=pl.BlockSpec((B, OUT), lambda i: (i, 0)),
        out_shape=jax.ShapeDtypeStruct((N, OUT), jnp.float32),
    )(onum, oden, x, Wo, bo, skip2)


def kernel(x, edge_index, edge_type, Wk, bk, Wq, bq, Wv, bv,
           a_rel, m_rel, p_rel, Wo, bo, skip):
    scale = (p_rel * (1.0 / np.sqrt(D))).astype(jnp.float32)
    Wk2 = jnp.einsum("ihd,rhdf->rihf", Wk.reshape(IN, H, D), a_rel)
    Wk2 = (Wk2 * scale[:, None, :, None]).reshape(2, IN, OUT)
    bk2 = jnp.einsum("hd,rhdf->rhf", bk.reshape(H, D), a_rel)
    bk2 = (bk2 * scale[:, :, None]).reshape(2, 1, OUT)
    Wv2 = jnp.einsum("ihd,rhdf->rihf", Wv.reshape(IN, H, D),
                     m_rel).reshape(2, IN, OUT)
    bv2 = jnp.einsum("hd,rhdf->rhf", bv.reshape(H, D),
                     m_rel).reshape(2, 1, OUT)

    src2 = edge_index[0].reshape(10, 1, EB)
    et2 = edge_type.reshape(10, 1, EB)
    qt, kcat, vcat, kvi2 = _front(x, Wq, bq.reshape(1, OUT), Wk2, bk2,
                                  Wv2, bv2, src2, et2)
    kt = kcat.reshape(2 * N, OUT)
    vt = vcat.reshape(2 * N, OUT)
    z64 = jnp.zeros((CH, OUT), jnp.float32)

    oacc = _sc_edges(qt, kt, vt, edge_index[1], kvi2.reshape(E), z64)
    onum = oacc[:, :N, :]
    oden = oacc[:, N:N + DROWS, :].reshape(2, N, 16)
    return _back(onum, oden, x, Wo, bo.reshape(1, OUT), skip.reshape(1, 1))


# speedup vs baseline: 5.8359x; 1.0077x over previous
import functools

import jax
import jax.numpy as jnp
import numpy as np
from jax import lax
from jax.experimental import pallas as pl
from jax.experimental.pallas import tpu as pltpu
from jax.experimental.pallas import tpu_sc as plsc

N = 10000
E = 320000
IN = 128
OUT = 128
H = 8
D = 16

CH = 32
NW = 32
NC2 = 157
NCHP = NC2 * 2 * NW
EP = NCHP * CH
NP = N + 80
DOFF = NP
DROWS = N // 8 + 11
R = 11360
ZROWS = 40
NZB = R // ZROWS
EB = E // 10


def _front_body(x_ref, wq_ref, bq_ref, wk_ref, bk_ref, wv_ref, bv_ref,
                src_ref, et_ref, q_ref, k_ref, v_ref, kv_ref):
    xb = x_ref[...]
    dot = functools.partial(jnp.dot, precision=lax.Precision.HIGHEST,
                            preferred_element_type=jnp.float32)
    q_ref[...] = dot(xb, wq_ref[...]) + bq_ref[...]
    for r in range(2):
        k_ref[r] = dot(xb, wk_ref[r]) + bk_ref[r]
        v_ref[r] = dot(xb, wv_ref[r]) + bv_ref[r]
    s = src_ref[...]
    kv_ref[...] = jnp.where(et_ref[...] == 1, s, s + N)


def _front(x, Wq, bq, Wk2, bk2, Wv2, bv2, src2, et2):
    B = 1000
    nb = N // B
    return pl.pallas_call(
        _front_body,
        grid=(nb,),
        in_specs=[
            pl.BlockSpec((B, IN), lambda i: (i, 0)),
            pl.BlockSpec((IN, OUT), lambda i: (0, 0)),
            pl.BlockSpec((1, OUT), lambda i: (0, 0)),
            pl.BlockSpec((2, IN, OUT), lambda i: (0, 0, 0)),
            pl.BlockSpec((2, 1, OUT), lambda i: (0, 0, 0)),
            pl.BlockSpec((2, IN, OUT), lambda i: (0, 0, 0)),
            pl.BlockSpec((2, 1, OUT), lambda i: (0, 0, 0)),
            pl.BlockSpec((1, 1, EB), lambda i: (i, 0, 0)),
            pl.BlockSpec((1, 1, EB), lambda i: (i, 0, 0)),
        ],
        out_specs=[
            pl.BlockSpec((B, OUT), lambda i: (i, 0)),
            pl.BlockSpec((2, B, OUT), lambda i: (0, i, 0)),
            pl.BlockSpec((2, B, OUT), lambda i: (0, i, 0)),
            pl.BlockSpec((1, 1, EB), lambda i: (i, 0, 0)),
        ],
        out_shape=[
            jax.ShapeDtypeStruct((N, OUT), jnp.float32),
            jax.ShapeDtypeStruct((2, N, OUT), jnp.float32),
            jax.ShapeDtypeStruct((2, N, OUT), jnp.float32),
            jax.ShapeDtypeStruct((10, 1, EB), jnp.int32),
        ],
    )(x, Wq, bq, Wk2, bk2, Wv2, bv2, src2, et2)


def _sc_body(qt, kt, vt, dst, kvi, z32, oacc,
             qb0, kb0, vb0, qb1, kb1, vb1, wb,
             dstb0, kvb0, dib0, dstb1, kvb1, dib1, acc_sh,
             sq0, sk0, sv0, sa0, sb0, sq1, sk1, sv1, sa1, sb1):
    cid = lax.axis_index("c")
    sid = lax.axis_index("s")
    wid = cid * 16 + sid
    i16 = lax.iota(jnp.int32, 16)
    zeros16 = jnp.zeros((16,), jnp.float32)

    sets = [
        (qb0, kb0, vb0, dstb0, kvb0, dib0, sq0, sk0, sv0, sa0, sb0),
        (qb1, kb1, vb1, dstb1, kvb1, dib1, sq1, sk1, sv1, sa1, sb1),
    ]

    def _zero_rows(i, _):
        for j in range(OUT // 16):
            vb0[i, pl.ds(j * 16, 16)] = zeros16
        return 0
    lax.fori_loop(0, CH, _zero_rows, 0)

    nz = jnp.where(sid < (NZB % 16), NZB // 16 + 1, NZB // 16)

    def _zstripe(i, _):
        r0 = (sid + i * 16) * ZROWS
        pltpu.sync_copy(vb0.at[pl.ds(0, CH)], acc_sh.at[pl.ds(r0, CH)])
        pltpu.sync_copy(vb0.at[pl.ds(0, ZROWS - CH)],
                        acc_sh.at[pl.ds(r0 + CH, ZROWS - CH)])
        return 0
    lax.fori_loop(0, nz, _zstripe, 0)
    plsc.subcore_barrier()

    def _idx_load(S, e):
        base = (wid + e * NW) * CH
        pltpu.sync_copy(dst.at[pl.ds(base, CH)], S[3])
        pltpu.sync_copy(kvi.at[pl.ds(base, CH)], S[4])

    def _gathers_start(S):
        return (pltpu.async_copy(qt.at[S[3]], S[0], S[6]),
                pltpu.async_copy(kt.at[S[4]], S[1], S[7]),
                pltpu.async_copy(vt.at[S[4]], S[2], S[8]))

    def _scatters_start(S):
        return (pltpu.async_copy(S[2], acc_sh.at[S[3]], S[9], add=True),
                pltpu.async_copy(S[0], acc_sh.at[S[5]], S[10], add=True))

    def _compute(S, g3):
        qb, kb, vb, dstb, kvb, dib = S[:6]
        g3[0].wait()
        g3[1].wait()

        def _alpha(gh, _):
            g = gh // H
            h = gh % H
            rows = g * 16 + i16
            acc = zeros16
            for d in range(D):
                col = jnp.broadcast_to(h * D + d, (16,)).astype(jnp.int32)
                qv = plsc.load_gather(qb, [rows, col])
                kv = plsc.load_gather(kb, [rows, col])
                acc = acc + qv * kv
            w16 = jnp.exp(acc)
            plsc.store_scatter(wb, [rows * 16 + h], w16)
            return 0
        lax.fori_loop(0, (CH // 16) * H, _alpha, 0)

        pltpu.sync_copy(z32, qb)

        def _den(g, _):
            rows = g * 16 + i16
            dst16 = dstb[pl.ds(g * 16, 16)]
            dib[pl.ds(g * 16, 16)] = DOFF + (dst16 >> 3)
            cbase = (dst16 & 7) * 16
            for h in range(H):
                w16 = plsc.load_gather(wb, [rows * 16 + h])
                plsc.store_scatter(qb, [rows, cbase + h], w16)
            return 0
        lax.fori_loop(0, CH // 16, _den, 0)

        g3[2].wait()

        def _msg(e, _):
            for h in range(H):
                wspl = plsc.load_gather(wb, [jnp.broadcast_to(
                    e * 16 + h, (16,)).astype(jnp.int32)])
                v16 = vb[e, pl.ds(h * D, 16)]
                vb[e, pl.ds(h * D, 16)] = v16 * wspl
            return 0
        lax.fori_loop(0, CH, _msg, 0)

    A, B = sets

    def _iter(c2, _):
        e0 = 2 * c2
        e1 = 2 * c2 + 1
        _idx_load(A, e0)
        ga = _gathers_start(A)
        _idx_load(B, e1)
        gb = _gathers_start(B)
        _compute(A, ga)
        sa = _scatters_start(A)
        _compute(B, gb)
        sa[0].wait()
        sa[1].wait()
        sb = _scatters_start(B)
        sb[0].wait()
        sb[1].wait()
        return 0

    lax.fori_loop(0, NC2, _iter, 0)
    plsc.subcore_barrier()

    def _fstripe(i, _):
        r0 = (sid + i * 16) * ZROWS
        pltpu.sync_copy(acc_sh.at[pl.ds(r0, ZROWS)],
                        oacc.at[cid, pl.ds(r0, ZROWS)])
        return 0
    lax.fori_loop(0, nz, _fstripe, 0)


@functools.partial(
    pl.kernel,
    out_type=jax.ShapeDtypeStruct((2, R, OUT), jnp.float32),
    mesh=plsc.VectorSubcoreMesh(core_axis_name="c", subcore_axis_name="s",
                                num_cores=2, num_subcores=16),
    compiler_params=pltpu.CompilerParams(needs_layout_passes=False),
    scratch_types=[
        pltpu.VMEM((CH, OUT), jnp.float32),
        pltpu.VMEM((CH, OUT), jnp.float32),
        pltpu.VMEM((CH, OUT), jnp.float32),
        pltpu.VMEM((CH, OUT), jnp.float32),
        pltpu.VMEM((CH, OUT), jnp.float32),
        pltpu.VMEM((CH, OUT), jnp.float32),
        pltpu.VMEM((CH * 16,), jnp.float32),
        pltpu.VMEM((CH,), jnp.int32),
        pltpu.VMEM((CH,), jnp.int32),
        pltpu.VMEM((CH,), jnp.int32),
        pltpu.VMEM((CH,), jnp.int32),
        pltpu.VMEM((CH,), jnp.int32),
        pltpu.VMEM((CH,), jnp.int32),
        pltpu.VMEM_SHARED((R, OUT), jnp.float32),
        pltpu.SemaphoreType.DMA,
        pltpu.SemaphoreType.DMA,
        pltpu.SemaphoreType.DMA,
        pltpu.SemaphoreType.DMA,
        pltpu.SemaphoreType.DMA,
        pltpu.SemaphoreType.DMA,
        pltpu.SemaphoreType.DMA,
        pltpu.SemaphoreType.DMA,
        pltpu.SemaphoreType.DMA,
        pltpu.SemaphoreType.DMA,
    ],
)
def _sc_edges(qt, kt, vt, dst, kvi, z32, oacc, *scratch):
    _sc_body(qt, kt, vt, dst, kvi, z32, oacc, *scratch)


def _back_body(n_ref, d_ref, x_ref, wo_ref, bo_ref, sk_ref, o_ref):
    num = n_ref[0] + n_ref[1]
    den = d_ref[0] + d_ref[1]
    rr = lax.broadcasted_iota(jnp.int32, (16, OUT), 0)
    cc = lax.broadcasted_iota(jnp.int32, (16, OUT), 1)
    expand = (cc // D == rr).astype(jnp.float32)
    den128 = jnp.dot(den, expand, precision=lax.Precision.HIGHEST,
                     preferred_element_type=jnp.float32)
    agg = num / (den128 + 1e-16)
    act = jax.nn.gelu(agg)
    out = jnp.dot(act, wo_ref[...], precision=lax.Precision.HIGHEST,
                  preferred_element_type=jnp.float32) + bo_ref[...]
    beta = jax.nn.sigmoid(sk_ref[0, 0])
    o_ref[...] = beta * out + (1.0 - beta) * x_ref[...]


def _back(onum, oden, x, Wo, bo, skip2):
    B = 1000
    nb = N // B
    return pl.pallas_call(
        _back_body,
        grid=(nb,),
        in_specs=[
            pl.BlockSpec((2, B, OUT), lambda i: (0, i, 0)),
            pl.BlockSpec((2, B, 16), lambda i: (0, i, 0)),
            pl.BlockSpec((B, IN), lambda i: (i, 0)),
            pl.BlockSpec((OUT, OUT), lambda i: (0, 0)),
            pl.BlockSpec((1, OUT), lambda i: (0, 0)),
            pl.BlockSpec((1, 1), lambda i: (0, 0)),
        ],
        out_specs=pl.BlockSpec((B, OUT), lambda i: (i, 0)),
        out_shape=jax.ShapeDtypeStruct((N, OUT), jnp.float32),
    )(onum, oden, x, Wo, bo, skip2)


def kernel(x, edge_index, edge_type, Wk, bk, Wq, bq, Wv, bv,
           a_rel, m_rel, p_rel, Wo, bo, skip):
    scale = (p_rel * (1.0 / np.sqrt(D))).astype(jnp.float32)
    Wk2 = jnp.einsum("ihd,rhdf->rihf", Wk.reshape(IN, H, D), a_rel)
    Wk2 = (Wk2 * scale[:, None, :, None]).reshape(2, IN, OUT)
    bk2 = jnp.einsum("hd,rhdf->rhf", bk.reshape(H, D), a_rel)
    bk2 = (bk2 * scale[:, :, None]).reshape(2, 1, OUT)
    Wv2 = jnp.einsum("ihd,rhdf->rihf", Wv.reshape(IN, H, D),
                     m_rel).reshape(2, IN, OUT)
    bv2 = jnp.einsum("hd,rhdf->rhf", bv.reshape(H, D),
                     m_rel).reshape(2, 1, OUT)

    src2 = edge_index[0].reshape(10, 1, EB)
    et2 = edge_type.reshape(10, 1, EB)
    qt, kcat, vcat, kvi2 = _front(x, Wq, bq.reshape(1, OUT), Wk2, bk2,
                                  Wv2, bv2, src2, et2)
    kt = kcat.reshape(2 * N, OUT)
    vt = vcat.reshape(2 * N, OUT)

    qtp = jnp.concatenate(
        [qt, jnp.zeros((NP - N, OUT), jnp.float32)], axis=0)
    dstp = jnp.concatenate(
        [edge_index[1], jnp.full((EP - E,), N, jnp.int32)])
    kvip = jnp.concatenate(
        [kvi2.reshape(E), jnp.zeros((EP - E,), jnp.int32)])
    z32 = jnp.zeros((CH, OUT), jnp.float32)

    oacc = _sc_edges(qtp, kt, vt, dstp, kvip, z32)
    onum = oacc[:, :N, :]
    oden = oacc[:, DOFF:DOFF + N // 8, :].reshape(2, N, 16)
    return _back(onum, oden, x, Wo, bo.reshape(1, OUT), skip.reshape(1, 1))
